# Initial kernel scaffold; baseline (speedup 1.0000x reference)
import jax
import jax.numpy as jnp
from jax.experimental import pallas as pl


def kernel(z, edge_index, r, t):
    raise NotImplementedError("write your pallas kernel here")


# speedup vs baseline: 1.3364x; 1.3364x over previous
import functools

import jax
import jax.numpy as jnp
from jax import lax
from jax.experimental import pallas as pl
from jax.experimental.pallas import tpu as pltpu
from jax.experimental.pallas import tpu_sc as plsc

_L = 16


def _sc_inner_products(z, src_idx, dst_idx):
    nc, ns = 2, 16
    nw = nc * ns
    e_total = src_idx.shape[0]
    d = z.shape[1]
    ew = e_total // nw
    assert ew * nw == e_total and ew % 8 == 0
    chunk = 80
    nchunk = ew // chunk
    assert nchunk * chunk == ew and chunk % _L == 0

    @functools.partial(
        pl.kernel,
        out_type=jax.ShapeDtypeStruct((e_total,), jnp.float32),
        mesh=plsc.VectorSubcoreMesh(core_axis_name="c", subcore_axis_name="s"),
        compiler_params=pltpu.CompilerParams(needs_layout_passes=False),
        scratch_types=[
            pltpu.VMEM((ew,), jnp.int32),
            pltpu.VMEM((ew,), jnp.int32),
            pltpu.VMEM((chunk, d), jnp.float32),
            pltpu.VMEM((chunk, d), jnp.float32),
            pltpu.VMEM((ew,), jnp.float32),
            pltpu.SemaphoreType.DMA,
            pltpu.SemaphoreType.DMA,
        ],
    )
    def body(z_hbm, sidx_hbm, didx_hbm, out_hbm,
             sidx_v, didx_v, srows_v, drows_v, out_v, sem_s, sem_d):
        wid = lax.axis_index("s") * nc + lax.axis_index("c")
        base = wid * ew
        pltpu.sync_copy(sidx_hbm.at[pl.ds(base, ew)], sidx_v)
        pltpu.sync_copy(didx_hbm.at[pl.ds(base, ew)], didx_v)
        lanes = lax.iota(jnp.int32, _L)

        def do_chunk(c, carry):
            cs = pltpu.make_async_copy(
                z_hbm.at[sidx_v.at[pl.ds(c * chunk, chunk)]], srows_v, sem_s)
            cd = pltpu.make_async_copy(
                z_hbm.at[didx_v.at[pl.ds(c * chunk, chunk)]], drows_v, sem_d)
            cs.start()
            cd.start()
            cs.wait()
            cd.wait()

            def do_group(g, carry2):
                rows = g * _L + lanes
                col0 = jnp.zeros((_L,), jnp.int32)
                s = plsc.load_gather(srows_v, [rows, col0])
                dd = plsc.load_gather(drows_v, [rows, col0])
                acc = -(s * dd)
                for f in range(1, d):
                    colf = jnp.full((_L,), f, jnp.int32)
                    s = plsc.load_gather(srows_v, [rows, colf])
                    dd = plsc.load_gather(drows_v, [rows, colf])
                    acc = acc + s * dd
                out_v[pl.ds(c * chunk + g * _L, _L)] = acc
                return carry2

            lax.fori_loop(0, chunk // _L, do_group, 0)
            return carry

        lax.fori_loop(0, nchunk, do_chunk, 0)
        pltpu.sync_copy(out_v, out_hbm.at[pl.ds(base, ew)])

    return body(z, src_idx, dst_idx)


def _tc_decode_body(r_ref, t_ref, inner_ref, o_ref):
    inner = inner_ref[...]
    arg = jnp.maximum(-inner, 1.0 + 1e-7)
    dist = jnp.log(arg + jnp.sqrt(arg * arg - 1.0))
    o_ref[...] = 1.0 / (jnp.exp((dist - r_ref[0, 0]) / t_ref[0, 0]) + 1.0)


def _tc_decode(inner2d, r, t):
    rows, cols = inner2d.shape
    return pl.pallas_call(
        _tc_decode_body,
        out_shape=jax.ShapeDtypeStruct((rows, cols), jnp.float32),
        in_specs=[
            pl.BlockSpec(memory_space=pltpu.SMEM),
            pl.BlockSpec(memory_space=pltpu.SMEM),
            pl.BlockSpec(memory_space=pltpu.VMEM),
        ],
        out_specs=pl.BlockSpec(memory_space=pltpu.VMEM),
    )(r.reshape(1, 1).astype(jnp.float32), t.reshape(1, 1).astype(jnp.float32),
      inner2d)


def kernel(z, edge_index, r, t):
    ei = edge_index.astype(jnp.int32)
    inner = _sc_inner_products(z, ei[0], ei[1])
    e_total = inner.shape[0]
    probs2d = _tc_decode(inner.reshape(e_total // 128, 128), r, t)
    return probs2d.reshape(e_total)
